# Initial kernel scaffold; baseline (speedup 1.0000x reference)
import jax
import jax.numpy as jnp
from jax.experimental import pallas as pl


def kernel(x, pe, dev):
    raise NotImplementedError("write your pallas kernel here")


# speedup vs baseline: 1.7311x; 1.7311x over previous
import functools

import jax
import jax.numpy as jnp
from jax import lax
from jax.experimental import pallas as pl
from jax.experimental.pallas import tpu as pltpu
from jax.experimental.pallas import tpu_sc as plsc

B = 16384
NF = 26
D = 64
TOT = B * NF
NC = 2
NS = 16
NW = NC * NS
PER_W = TOT // NW
CHUNK = 128
NCHUNK = PER_W // CHUNK

_mesh = plsc.VectorSubcoreMesh(
    core_axis_name="c", subcore_axis_name="s", num_cores=NC, num_subcores=NS
)


@functools.partial(
    pl.kernel,
    out_type=jax.ShapeDtypeStruct((TOT, D), jnp.float32),
    mesh=_mesh,
    scratch_types=[
        pltpu.VMEM((NCHUNK, CHUNK), jnp.int32),
        pltpu.VMEM((CHUNK, D), jnp.float32),
        pltpu.SemaphoreType.DMA,
    ],
    compiler_params=pltpu.CompilerParams(use_tc_tiling_on_sc=False),
)
def _gather_kernel(pe_hbm, idx_hbm, out_hbm, idx_v, rows_v, sem):
    wid = lax.axis_index("s") * NC + lax.axis_index("c")
    base = wid * PER_W
    pltpu.sync_copy(idx_hbm.at[wid], idx_v)

    def step(g, carry):
        pltpu.async_copy(pe_hbm.at[idx_v.at[g]], rows_v, sem).wait()
        pltpu.sync_copy(rows_v, out_hbm.at[pl.ds(base + g * CHUNK, CHUNK)])
        return carry

    lax.fori_loop(0, NCHUNK, step, 0)


def kernel(x, pe, dev=0):
    idx = x.reshape(NW, NCHUNK, CHUNK)
    out = _gather_kernel(pe, idx)
    return out.reshape(B, NF * D)


# speedup vs baseline: 2.1863x; 1.2630x over previous
import functools

import jax
import jax.numpy as jnp
from jax import lax
from jax.experimental import pallas as pl
from jax.experimental.pallas import tpu as pltpu
from jax.experimental.pallas import tpu_sc as plsc

B = 16384
NF = 26
D = 64
TOT = B * NF
NC = 2
NS = 16
NW = NC * NS
PER_W = TOT // NW
CHUNK = 128
NCHUNK = PER_W // CHUNK
NBUF = 8
NSTEP = NCHUNK // NBUF

_mesh = plsc.VectorSubcoreMesh(
    core_axis_name="c", subcore_axis_name="s", num_cores=NC, num_subcores=NS
)


@functools.partial(
    pl.kernel,
    out_type=jax.ShapeDtypeStruct((TOT, D), jnp.float32),
    mesh=_mesh,
    scratch_types=[
        pltpu.VMEM((NCHUNK, CHUNK), jnp.int32),
        pltpu.VMEM((NBUF, CHUNK, D), jnp.float32),
        pltpu.SemaphoreType.DMA((NBUF,)),
        pltpu.SemaphoreType.DMA((NBUF,)),
    ],
    compiler_params=pltpu.CompilerParams(use_tc_tiling_on_sc=False),
)
def _gather_kernel(pe_hbm, idx_hbm, out_hbm, idx_v, rows_v, sem_in, sem_out):
    wid = lax.axis_index("s") * NC + lax.axis_index("c")
    base = wid * PER_W
    pltpu.sync_copy(idx_hbm.at[wid], idx_v)

    def gather_start(g, b):
        pltpu.async_copy(pe_hbm.at[idx_v.at[g]], rows_v.at[b], sem_in.at[b])

    def gather_wait(b):
        pltpu.make_async_copy(pe_hbm.at[idx_v.at[0]], rows_v.at[b], sem_in.at[b]).wait()

    def wb_start(g, b):
        pltpu.async_copy(
            rows_v.at[b], out_hbm.at[pl.ds(base + g * CHUNK, CHUNK)], sem_out.at[b]
        )

    def wb_wait(b):
        pltpu.make_async_copy(
            rows_v.at[b], out_hbm.at[pl.ds(base, CHUNK)], sem_out.at[b]
        ).wait()

    for b in range(NBUF):
        gather_start(b, b)

    def step(j, carry):
        for b in range(NBUF):
            gather_wait(b)
            wb_start(j * NBUF + b, b)
        for b in range(NBUF):
            wb_wait(b)
            gather_start((j + 1) * NBUF + b, b)
        return carry

    lax.fori_loop(0, NSTEP - 1, step, 0)

    for b in range(NBUF):
        gather_wait(b)
        wb_start((NSTEP - 1) * NBUF + b, b)
    for b in range(NBUF):
        wb_wait(b)


def kernel(x, pe, dev=0):
    idx = x.reshape(NW, NCHUNK, CHUNK)
    out = _gather_kernel(pe, idx)
    return out.reshape(B, NF * D)
